# baseline (speedup 1.0000x reference)
import functools

import jax
import jax.numpy as jnp
from jax import lax
from jax.experimental import pallas as pl
from jax.experimental.pallas import tpu as pltpu
from jax.experimental.pallas import tpu_sc as plsc

ALPHA = 0.6
BETA = 0.2
GAMMA = 0.2
MASKING_RATIO = 0.5

LANES = 128
SC_LANES = 16


def _score_body(x_ref, g_ref, p_ref, s_ref):
    x = x_ref[0]
    e = jnp.sqrt(jnp.sum(x * x, axis=0))
    g = jax.nn.sigmoid(g_ref[0, 0])
    p = jax.nn.sigmoid(p_ref[0, 0])
    s_ref[0, 0] = ALPHA * e + BETA * g + GAMMA * p


def _sc_energy_body(body_c, body_hs, body_ht, x_hbm, e_hbm):
    C = body_c
    HS = body_hs
    HT = body_ht
    W = x_hbm.shape[3]
    B = x_hbm.shape[0]

    def body(x_vmem, e_vmem):
        @pl.loop(0, W, step=SC_LANES)
        def _(w0):
            acc = [jnp.zeros((SC_LANES,), jnp.float32) for _ in range(4)]
            for c in range(C):
                v = x_vmem[0, c, 0, pl.ds(w0, SC_LANES)]
                acc[c % 4] = acc[c % 4] + v * v
            e_vmem[0, 0, 0, pl.ds(w0, SC_LANES)] = (acc[0] + acc[1]) + (
                acc[2] + acc[3]
            )

    pltpu.emit_pipeline(
        body,
        grid=(B * HS,),
        in_specs=[
            pl.BlockSpec(
                (1, C, 1, W), index_map=lambda i: (i // HS, 0, HT + (i % HS), 0)
            )
        ],
        out_specs=[
            pl.BlockSpec((1, 1, 1, W), index_map=lambda i: (i // HS, 0, i % HS, 0))
        ],
        core_axis_name=("c", "s"),
        dimension_semantics=(pltpu.PARALLEL,),
    )(x_hbm, e_hbm)


def _select_body(s1_ref, e2_ref, g2_ref, p2_ref, m_ref, *, k):
    s1 = s1_ref[...]
    e2 = e2_ref[...]
    s2 = (
        ALPHA * jnp.sqrt(e2)
        + BETA * jax.nn.sigmoid(g2_ref[...])
        + GAMMA * jax.nn.sigmoid(p2_ref[...])
    )
    s = jnp.concatenate([s1, s2], axis=1)
    B, NR, L = s.shape
    n = NR * L
    bits = lax.bitcast_convert_type(s, jnp.int32)

    def count_ge(t):
        return jnp.sum((bits >= t).astype(jnp.int32), axis=(1, 2), keepdims=True)

    lo = jnp.zeros((B, 1, 1), jnp.int32)
    hi = jnp.full((B, 1, 1), 0x7F800000, jnp.int32)

    def bs_body(_, lohi):
        lo, hi = lohi
        mid = lo + (hi - lo + 1) // 2
        pred = count_ge(mid) >= k
        return jnp.where(pred, mid, lo), jnp.where(pred, hi, mid - 1)

    lo, hi = lax.fori_loop(0, 31, bs_body, (lo, hi))
    t = lo

    gt = bits > t
    cnt_gt = jnp.sum(gt.astype(jnp.int32), axis=(1, 2), keepdims=True)
    need = k - cnt_gt
    tie = bits == t
    idx = (
        lax.broadcasted_iota(jnp.int32, (B, NR, L), 1) * L
        + lax.broadcasted_iota(jnp.int32, (B, NR, L), 2)
    )

    lo_j = jnp.zeros((B, 1, 1), jnp.int32)
    hi_j = jnp.full((B, 1, 1), n, jnp.int32)

    def bs2_body(_, lohi):
        lo, hi = lohi
        mid = (lo + hi) // 2
        g = jnp.sum((tie & (idx < mid)).astype(jnp.int32), axis=(1, 2), keepdims=True)
        pred = g >= need
        return jnp.where(pred, lo, mid + 1), jnp.where(pred, mid, hi)

    lo_j, hi_j = lax.fori_loop(0, 18, bs2_body, (lo_j, hi_j))
    j = lo_j

    m_ref[...] = (gt | (tie & (idx < j))).astype(jnp.float32)


def _apply_body(x_ref, m_ref, o_ref):
    o_ref[...] = x_ref[...] * m_ref[...]


def kernel(x, gaze_importance, pose_importance):
    B, C, H, W = x.shape
    N = H * W
    k = int(MASKING_RATIO * N)

    HS = H // 3
    HT = H - HS
    NBT = 4 if HT % 4 == 0 else 1
    BH = HT // NBT
    NB = 6 if H % 6 == 0 else 1
    BHA = H // NB

    gi4 = gaze_importance.reshape(B, 1, H, W)
    pi4 = pose_importance.reshape(B, 1, H, W)

    mesh = plsc.VectorSubcoreMesh(core_axis_name="c", subcore_axis_name="s")
    energy_sc = pl.kernel(
        functools.partial(_sc_energy_body, C, HS, HT),
        mesh=mesh,
        out_type=jax.ShapeDtypeStruct((B, 1, HS, W), jnp.float32),
    )(x)

    if True:
        e_tc = jnp.sum(x[:, :, :HT, :] * x[:, :, :HT, :], axis=1, keepdims=True)
        scores_tc = (
            ALPHA * jnp.sqrt(e_tc)
            + BETA * jax.nn.sigmoid(gi4[:, :, :HT, :])
            + GAMMA * jax.nn.sigmoid(pi4[:, :, :HT, :])
        )
    else:
        scores_tc = pl.pallas_call(
            _score_body,
            grid=(B, NBT),
            in_specs=[
                pl.BlockSpec((1, C, BH, W), lambda b, i: (b, 0, i, 0)),
                pl.BlockSpec((1, 1, BH, W), lambda b, i: (b, 0, i, 0)),
                pl.BlockSpec((1, 1, BH, W), lambda b, i: (b, 0, i, 0)),
            ],
            out_specs=pl.BlockSpec((1, 1, BH, W), lambda b, i: (b, 0, i, 0)),
            out_shape=jax.ShapeDtypeStruct((B, 1, HT, W), jnp.float32),
        )(x, gi4, pi4)

    NR1 = HT * W // LANES
    NR2 = HS * W // LANES
    s1 = scores_tc.reshape(B, NR1, LANES)
    e2 = energy_sc.reshape(B, NR2, LANES)
    g2 = gi4[:, :, HT:, :].reshape(B, NR2, LANES)
    p2 = pi4[:, :, HT:, :].reshape(B, NR2, LANES)

    NR = N // LANES
    mask = pl.pallas_call(
        functools.partial(_select_body, k=k),
        in_specs=[
            pl.BlockSpec((B, NR1, LANES), lambda: (0, 0, 0)),
            pl.BlockSpec((B, NR2, LANES), lambda: (0, 0, 0)),
            pl.BlockSpec((B, NR2, LANES), lambda: (0, 0, 0)),
            pl.BlockSpec((B, NR2, LANES), lambda: (0, 0, 0)),
        ],
        out_specs=pl.BlockSpec((B, NR, LANES), lambda: (0, 0, 0)),
        out_shape=jax.ShapeDtypeStruct((B, NR, LANES), jnp.float32),
    )(s1, e2, g2, p2)
    mask = mask.reshape(B, 1, H, W)

    out = pl.pallas_call(
        _apply_body,
        grid=(B, NB),
        in_specs=[
            pl.BlockSpec((1, C, BHA, W), lambda b, i: (b, 0, i, 0)),
            pl.BlockSpec((1, 1, BHA, W), lambda b, i: (b, 0, i, 0)),
        ],
        out_specs=pl.BlockSpec((1, C, BHA, W), lambda b, i: (b, 0, i, 0)),
        out_shape=jax.ShapeDtypeStruct((B, C, H, W), jnp.float32),
    )(x, mask)

    return out


# speedup vs baseline: 1.0825x; 1.0825x over previous
import functools

import jax
import jax.numpy as jnp
from jax import lax
from jax.experimental import pallas as pl
from jax.experimental.pallas import tpu as pltpu

ALPHA = 0.6
BETA = 0.2
GAMMA = 0.2
MASKING_RATIO = 0.5

LANES = 128


def _score_body(x_ref, g_ref, p_ref, s_ref):
    x = x_ref[0]
    e = jnp.sqrt(jnp.sum(x * x, axis=0))
    g = jax.nn.sigmoid(g_ref[0, 0])
    p = jax.nn.sigmoid(p_ref[0, 0])
    s_ref[0, 0] = ALPHA * e + BETA * g + GAMMA * p


def _select_body(s_ref, m_ref, *, k):
    s = s_ref[...]
    B, NR, L = s.shape
    n = NR * L
    bits = lax.bitcast_convert_type(s, jnp.int32)

    def count_ge(t):
        return jnp.sum((bits >= t).astype(jnp.int32), axis=(1, 2), keepdims=True)

    lo = jnp.zeros((B, 1, 1), jnp.int32)
    hi = jnp.full((B, 1, 1), 0x7F800000, jnp.int32)

    def bs_body(_, lohi):
        lo, hi = lohi
        mid = lo + (hi - lo + 1) // 2
        pred = count_ge(mid) >= k
        return jnp.where(pred, mid, lo), jnp.where(pred, hi, mid - 1)

    lo, hi = lax.fori_loop(0, 31, bs_body, (lo, hi))
    t = lo

    gt = bits > t
    cnt_gt = jnp.sum(gt.astype(jnp.int32), axis=(1, 2), keepdims=True)
    need = k - cnt_gt
    tie = bits == t
    idx = (
        lax.broadcasted_iota(jnp.int32, (B, NR, L), 1) * L
        + lax.broadcasted_iota(jnp.int32, (B, NR, L), 2)
    )

    lo_j = jnp.zeros((B, 1, 1), jnp.int32)
    hi_j = jnp.full((B, 1, 1), n, jnp.int32)

    def bs2_body(_, lohi):
        lo, hi = lohi
        mid = (lo + hi) // 2
        g = jnp.sum((tie & (idx < mid)).astype(jnp.int32), axis=(1, 2), keepdims=True)
        pred = g >= need
        return jnp.where(pred, lo, mid + 1), jnp.where(pred, mid, hi)

    lo_j, hi_j = lax.fori_loop(0, 18, bs2_body, (lo_j, hi_j))
    j = lo_j

    m_ref[...] = (gt | (tie & (idx < j))).astype(jnp.float32)


def _apply_body(x_ref, m_ref, o_ref):
    o_ref[...] = x_ref[...] * m_ref[...]


def kernel(x, gaze_importance, pose_importance):
    B, C, H, W = x.shape
    N = H * W
    k = int(MASKING_RATIO * N)

    NB = 6
    BH = H // NB
    NBA = 4
    BHA = H // NBA

    gi4 = gaze_importance.reshape(B, 1, H, W)
    pi4 = pose_importance.reshape(B, 1, H, W)
    scores = pl.pallas_call(
        _score_body,
        grid=(B, NB),
        in_specs=[
            pl.BlockSpec((1, C, BH, W), lambda b, i: (b, 0, i, 0)),
            pl.BlockSpec((1, 1, BH, W), lambda b, i: (b, 0, i, 0)),
            pl.BlockSpec((1, 1, BH, W), lambda b, i: (b, 0, i, 0)),
        ],
        out_specs=pl.BlockSpec((1, 1, BH, W), lambda b, i: (b, 0, i, 0)),
        out_shape=jax.ShapeDtypeStruct((B, 1, H, W), jnp.float32),
    )(x, gi4, pi4)

    NR = N // LANES
    mask = pl.pallas_call(
        functools.partial(_select_body, k=k),
        in_specs=[pl.BlockSpec((B, NR, LANES), lambda: (0, 0, 0))],
        out_specs=pl.BlockSpec((B, NR, LANES), lambda: (0, 0, 0)),
        out_shape=jax.ShapeDtypeStruct((B, NR, LANES), jnp.float32),
    )(scores.reshape(B, NR, LANES))
    mask = mask.reshape(B, 1, H, W)

    out = pl.pallas_call(
        _apply_body,
        grid=(B, NBA),
        in_specs=[
            pl.BlockSpec((1, C, BHA, W), lambda b, i: (b, 0, i, 0)),
            pl.BlockSpec((1, 1, BHA, W), lambda b, i: (b, 0, i, 0)),
        ],
        out_specs=pl.BlockSpec((1, C, BHA, W), lambda b, i: (b, 0, i, 0)),
        out_shape=jax.ShapeDtypeStruct((B, C, H, W), jnp.float32),
    )(x, mask)

    return out
